# baseline (speedup 1.0000x reference)
import functools

import jax
import jax.numpy as jnp
from jax import lax
from jax.experimental import pallas as pl
from jax.experimental.pallas import tpu as pltpu
from jax.experimental.pallas import tpu_sc as plsc

_ATTR_INDEX = 8

_NC = 2
_NS = 16
_L = 16
_NW = _NC * _NS
_CH = 256
_BB = 4096


def _tc_index_body(z_ref, o_ref):
    o_ref[:] = z_ref[:, _ATTR_INDEX].astype(jnp.int32)


def _make_tc_index(B, D):
    return pl.pallas_call(
        _tc_index_body,
        grid=(B // _BB,),
        in_specs=[pl.BlockSpec((_BB, D), lambda i: (i, 0))],
        out_specs=pl.BlockSpec((_BB,), lambda i: (i,)),
        out_shape=jax.ShapeDtypeStruct((B,), jnp.int32),
    )


def _make_sc_gather(B, C):
    n = B // _NW
    nch = n // _CH

    mesh = plsc.VectorSubcoreMesh(core_axis_name="c", subcore_axis_name="s")

    @functools.partial(
        pl.kernel,
        mesh=mesh,
        out_type=jax.ShapeDtypeStruct((B,), jnp.float32),
        compiler_params=pltpu.CompilerParams(needs_layout_passes=False),
        scratch_types=[
            pltpu.VMEM((2 * _CH, C), jnp.float32),
            pltpu.VMEM((n,), jnp.int32),
            pltpu.VMEM((n,), jnp.float32),
            pltpu.SemaphoreType.DMA,
        ],
    )
    def k(cidx_hbm, a_hbm, out_hbm, abuf, cbuf, obuf, asem):
        wid = lax.axis_index("s") * _NC + lax.axis_index("c")
        base = wid * n

        pltpu.sync_copy(cidx_hbm.at[pl.ds(base, n)], cbuf)

        iota = lax.iota(jnp.int32, _L)

        def a_copy(ch):
            return pltpu.make_async_copy(
                a_hbm.at[pl.ds(base + ch * _CH, _CH)],
                abuf.at[pl.ds((ch % 2) * _CH, _CH)],
                asem,
            )

        a_copy(0).start()
        for ch in range(nch):
            a_copy(ch).wait()
            if ch + 1 < nch:
                a_copy(ch + 1).start()
            par = (ch % 2) * _CH

            def extract(j, carry, ch=ch, par=par):
                rows = j * _L + iota
                cols = cbuf[pl.ds(ch * _CH + j * _L, _L)]
                vals = plsc.load_gather(abuf, [par + rows, cols])
                obuf[pl.ds(ch * _CH + j * _L, _L)] = vals * jnp.float32(0.999)
                return carry

            lax.fori_loop(0, _CH // _L, extract, 0)

        pltpu.sync_copy(obuf, out_hbm.at[pl.ds(base, n)])

    return k


@jax.jit
def kernel(z, a):
    b, c = a.shape
    cidx = _make_tc_index(b, z.shape[1])(z)
    return _make_sc_gather(b, c)(cidx, a)


# speedup vs baseline: 1.1836x; 1.1836x over previous
import functools

import jax
import jax.numpy as jnp
from jax import lax
from jax.experimental import pallas as pl
from jax.experimental.pallas import tpu as pltpu
from jax.experimental.pallas import tpu_sc as plsc

_ATTR_INDEX = 8

_NC = 2
_NS = 16
_L = 16
_NW = _NC * _NS
_CH = 128


def _make_sc_kernel(B, D, C):
    n = B // _NW
    nch = n // _CH

    mesh = plsc.VectorSubcoreMesh(core_axis_name="c", subcore_axis_name="s")

    @functools.partial(
        pl.kernel,
        mesh=mesh,
        out_type=jax.ShapeDtypeStruct((B,), jnp.float32),
        compiler_params=pltpu.CompilerParams(needs_layout_passes=False),
        scratch_types=[
            pltpu.VMEM((2 * _CH, D), jnp.float32),
            pltpu.VMEM((2 * _CH, C), jnp.float32),
            pltpu.VMEM((n,), jnp.float32),
            pltpu.SemaphoreType.DMA,
            pltpu.SemaphoreType.DMA,
        ],
    )
    def k(z_hbm, a_hbm, out_hbm, zbuf, abuf, obuf, zsem, asem):
        wid = lax.axis_index("s") * _NC + lax.axis_index("c")
        base = wid * n

        iota = lax.iota(jnp.int32, _L)
        col = jnp.full((_L,), _ATTR_INDEX, jnp.int32)

        def z_copy(ch):
            return pltpu.make_async_copy(
                z_hbm.at[pl.ds(base + ch * _CH, _CH)],
                zbuf.at[pl.ds((ch % 2) * _CH, _CH)],
                zsem,
            )

        def a_copy(ch):
            return pltpu.make_async_copy(
                a_hbm.at[pl.ds(base + ch * _CH, _CH)],
                abuf.at[pl.ds((ch % 2) * _CH, _CH)],
                asem,
            )

        z_copy(0).start()
        a_copy(0).start()
        for ch in range(nch):
            z_copy(ch).wait()
            a_copy(ch).wait()
            if ch + 1 < nch:
                z_copy(ch + 1).start()
                a_copy(ch + 1).start()
            par = (ch % 2) * _CH

            def extract(j, carry, ch=ch, par=par):
                rows = par + j * _L + iota
                zv = plsc.load_gather(zbuf, [rows, col])
                idxv = zv.astype(jnp.int32)
                av = plsc.load_gather(abuf, [rows, idxv])
                obuf[pl.ds(ch * _CH + j * _L, _L)] = av * jnp.float32(0.999)
                return carry

            lax.fori_loop(0, _CH // _L, extract, 0)

        pltpu.sync_copy(obuf, out_hbm.at[pl.ds(base, n)])

    return k


@jax.jit
def kernel(z, a):
    b, c = a.shape
    return _make_sc_kernel(b, z.shape[1], c)(z, a)


# speedup vs baseline: 2.3637x; 1.9971x over previous
import functools

import jax
import jax.numpy as jnp
from jax import lax
from jax.experimental import pallas as pl
from jax.experimental.pallas import tpu as pltpu
from jax.experimental.pallas import tpu_sc as plsc

_ATTR_INDEX = 8

_NC = 2
_NS = 16
_L = 16
_NW = _NC * _NS
_CH = 256
_NBUF = 3


def _make_sc_kernel(B, C):
    n = B // _NW
    nch = n // _CH

    mesh = plsc.VectorSubcoreMesh(core_axis_name="c", subcore_axis_name="s")

    @functools.partial(
        pl.kernel,
        mesh=mesh,
        out_type=jax.ShapeDtypeStruct((B,), jnp.float32),
        compiler_params=pltpu.CompilerParams(needs_layout_passes=False),
        scratch_types=[
            pltpu.VMEM((_NBUF * _CH, C), jnp.float32),
            pltpu.VMEM((n,), jnp.float32),
            pltpu.VMEM((n,), jnp.float32),
            pltpu.SemaphoreType.DMA,
        ],
    )
    def k(zcol_hbm, a_hbm, out_hbm, abuf, cbuf, obuf, asem):
        wid = lax.axis_index("s") * _NC + lax.axis_index("c")
        base = wid * n

        iota = lax.iota(jnp.int32, _L)

        def a_copy(ch):
            return pltpu.make_async_copy(
                a_hbm.at[pl.ds(base + ch * _CH, _CH)],
                abuf.at[pl.ds((ch % _NBUF) * _CH, _CH)],
                asem,
            )

        for ch in range(min(_NBUF - 1, nch)):
            a_copy(ch).start()
        pltpu.sync_copy(zcol_hbm.at[pl.ds(base, n)], cbuf)

        for ch in range(nch):
            a_copy(ch).wait()
            if ch + _NBUF - 1 < nch:
                a_copy(ch + _NBUF - 1).start()
            par = (ch % _NBUF) * _CH

            def extract(j, carry, ch=ch, par=par):
                rows = par + j * _L + iota
                cols = cbuf[pl.ds(ch * _CH + j * _L, _L)].astype(jnp.int32)
                av = plsc.load_gather(abuf, [rows, cols])
                obuf[pl.ds(ch * _CH + j * _L, _L)] = av * jnp.float32(0.999)
                return carry

            lax.fori_loop(0, _CH // _L, extract, 0)

        pltpu.sync_copy(obuf, out_hbm.at[pl.ds(base, n)])

    return k


@jax.jit
def kernel(z, a):
    b, c = a.shape
    zcol = z[:, _ATTR_INDEX]
    return _make_sc_kernel(b, c)(zcol, a)
